# baseline (speedup 1.0000x reference)
import functools

import jax
import jax.numpy as jnp
from jax import lax
from jax.experimental import pallas as pl
from jax.experimental.pallas import tpu as pltpu
from jax.experimental.pallas import tpu_sc as plsc

_ROWS = 4096 * 200
_NW = 32
_RPW = _ROWS // _NW
_R = 800
_NCHUNK = _RPW // _R
_G = _R * 2 // 16


def _body(x_hbm, out_hbm, inbuf):
    wid = lax.axis_index("s") * 2 + lax.axis_index("c")
    base = wid * _RPW

    def chunk(k, carry):
        row0 = base + k * _R
        pltpu.sync_copy(x_hbm.at[pl.ds(row0, _R)], inbuf)
        pltpu.sync_copy(inbuf.at[:, pl.ds(0, 2)], out_hbm.at[pl.ds(row0, _R)])
        return carry

    lax.fori_loop(0, _NCHUNK, chunk, 0)


def kernel(x):
    b, s, c = x.shape
    x1 = x.reshape(b * s, c)
    mesh = plsc.VectorSubcoreMesh(core_axis_name="c", subcore_axis_name="s")
    run = functools.partial(
        pl.kernel,
        out_type=jax.ShapeDtypeStruct((b * s, 2), jnp.float32),
        mesh=mesh,
        scratch_types=[
            pltpu.VMEM((_R, 128), jnp.float32),
        ],
        compiler_params=pltpu.CompilerParams(use_tc_tiling_on_sc=False),
    )(_body)
    return run(x1).reshape(b, s, 2)
